# baseline (speedup 1.0000x reference)
import functools

import jax
import jax.numpy as jnp
from jax import lax
from jax.experimental import pallas as pl
from jax.experimental.pallas import tpu as pltpu
from jax.experimental.pallas import tpu_sc as plsc

FACTOR = 32
BATCH = 16384
LANES = 16
CHUNK = 128

NC, NS = 2, 16
NW = NC * NS
B_PER_W = BATCH // NW
NCHUNK = B_PER_W // CHUNK
NGROUP = B_PER_W // LANES


def _gmf_body(user_hbm, item_hbm, tab_u, tab_i, w_hbm, b_hbm, out_hbm,
              idx_u, idx_i, vals_u, vals_i, w_v, b_v, out_v, sem):
    wid = lax.axis_index("s") * NC + lax.axis_index("c")
    base = wid * B_PER_W

    pltpu.sync_copy(user_hbm.at[wid], idx_u)
    pltpu.sync_copy(item_hbm.at[wid], idx_i)
    pltpu.sync_copy(w_hbm, w_v)
    pltpu.sync_copy(b_hbm, b_v)

    copies = []
    for f in range(FACTOR):
        for j in range(NCHUNK):
            dst = pl.ds(j * CHUNK, CHUNK)
            src_idx_u = idx_u.at[pl.ds(j * CHUNK, CHUNK)]
            src_idx_i = idx_i.at[pl.ds(j * CHUNK, CHUNK)]
            copies.append(
                pltpu.async_copy(tab_u.at[f].at[src_idx_u], vals_u.at[f, dst], sem))
            copies.append(
                pltpu.async_copy(tab_i.at[f].at[src_idx_i], vals_i.at[f, dst], sem))
    for c in copies:
        c.wait()

    bias = b_v[...]

    def group(g, carry):
        db = pl.ds(g * LANES, LANES)
        acc = bias
        for f in range(FACTOR):
            acc = acc + vals_u[f, db] * vals_i[f, db] * w_v[f, :]
        out_v[db] = acc
        return carry

    lax.fori_loop(0, NGROUP, group, 0)
    pltpu.sync_copy(out_v, out_hbm.at[pl.ds(base, B_PER_W)])


_gmf = functools.partial(
    pl.kernel,
    mesh=plsc.VectorSubcoreMesh(
        core_axis_name="c", subcore_axis_name="s",
        num_cores=NC, num_subcores=NS),
    out_type=jax.ShapeDtypeStruct((BATCH,), jnp.float32),
    compiler_params=pltpu.CompilerParams(
        needs_layout_passes=False, use_tc_tiling_on_sc=False),
    scratch_types=[
        pltpu.VMEM((B_PER_W,), jnp.int32),
        pltpu.VMEM((B_PER_W,), jnp.int32),
        pltpu.VMEM((FACTOR, B_PER_W), jnp.float32),
        pltpu.VMEM((FACTOR, B_PER_W), jnp.float32),
        pltpu.VMEM((FACTOR, LANES), jnp.float32),
        pltpu.VMEM((LANES,), jnp.float32),
        pltpu.VMEM((B_PER_W,), jnp.float32),
        pltpu.SemaphoreType.DMA,
    ],
)(_gmf_body)


def kernel(user, item, embed_user_GMF, embed_item_GMF, predict_W, predict_b):
    user_r = user.astype(jnp.int32).reshape(NW, B_PER_W)
    item_r = item.astype(jnp.int32).reshape(NW, B_PER_W)
    w_b = jnp.broadcast_to(predict_W.reshape(FACTOR, 1), (FACTOR, LANES))
    b_b = jnp.broadcast_to(predict_b.reshape(1), (LANES,))
    return _gmf(user_r, item_r, embed_user_GMF.T, embed_item_GMF.T, w_b, b_b)


# speedup vs baseline: 5.6032x; 5.6032x over previous
import functools

import jax
import jax.numpy as jnp
from jax import lax
from jax.experimental import pallas as pl
from jax.experimental.pallas import tpu as pltpu
from jax.experimental.pallas import tpu_sc as plsc

FACTOR = 32
BATCH = 16384
LANES = 16
CHUNK = 128
PACK = 4
PACKED_ROWS = 1000000 // PACK

NC, NS = 2, 16
NW = NC * NS
B_PER_W = BATCH // NW
NCHUNK = B_PER_W // CHUNK
GPC = CHUNK // LANES


def _gmf_body(user_hbm, item_hbm, tab_u, tab_i, w_hbm, b_hbm, out_hbm,
              idx_u, idx_i, pidx_u, pidx_i, rows_u, rows_i,
              w_v, b_v, out_v, sems):
    wid = lax.axis_index("s") * NC + lax.axis_index("c")
    base = wid * B_PER_W

    pltpu.sync_copy(user_hbm.at[wid], idx_u)
    pltpu.sync_copy(item_hbm.at[wid], idx_i)
    pltpu.sync_copy(w_hbm, w_v)
    pltpu.sync_copy(b_hbm, b_v)

    def shift(g, carry):
        cs = pl.ds(g * LANES, LANES)
        pidx_u[cs] = lax.shift_right_logical(idx_u[cs], 2)
        pidx_i[cs] = lax.shift_right_logical(idx_i[cs], 2)
        return carry

    lax.fori_loop(0, B_PER_W // LANES, shift, 0)

    def fire(j):
        slot = j % 2
        pltpu.async_copy(
            tab_u.at[pidx_u.at[pl.ds(j * CHUNK, CHUNK)]],
            rows_u.at[slot], sems.at[slot])
        pltpu.async_copy(
            tab_i.at[pidx_i.at[pl.ds(j * CHUNK, CHUNK)]],
            rows_i.at[slot], sems.at[slot])

    def drain(j):
        slot = j % 2
        pltpu.make_async_copy(
            tab_u.at[pidx_u.at[pl.ds(0, CHUNK)]],
            rows_u.at[slot], sems.at[slot]).wait()
        pltpu.make_async_copy(
            tab_i.at[pidx_i.at[pl.ds(0, CHUNK)]],
            rows_i.at[slot], sems.at[slot]).wait()

    bias = b_v[...]
    lidx = lax.iota(jnp.int32, LANES)

    def compute(j):
        slot = j % 2
        ru, ri = rows_u.at[slot], rows_i.at[slot]

        def group(g, carry):
            db = pl.ds(j * CHUNK + g * LANES, LANES)
            rows16 = g * LANES + lidx
            col_u = (idx_u[db] & (PACK - 1)) * FACTOR
            col_i = (idx_i[db] & (PACK - 1)) * FACTOR
            acc = bias
            for f in range(FACTOR):
                u = plsc.load_gather(ru, [rows16, col_u + f])
                v = plsc.load_gather(ri, [rows16, col_i + f])
                acc = acc + u * v * w_v[f, :]
            out_v[db] = acc
            return carry

        lax.fori_loop(0, GPC, group, 0)

    fire(0)
    fire(1)
    for j in range(NCHUNK):
        drain(j)
        compute(j)
        if j + 2 < NCHUNK:
            fire(j + 2)
    pltpu.sync_copy(out_v, out_hbm.at[pl.ds(base, B_PER_W)])


_gmf = functools.partial(
    pl.kernel,
    mesh=plsc.VectorSubcoreMesh(
        core_axis_name="c", subcore_axis_name="s",
        num_cores=NC, num_subcores=NS),
    out_type=jax.ShapeDtypeStruct((BATCH,), jnp.float32),
    compiler_params=pltpu.CompilerParams(
        needs_layout_passes=False, use_tc_tiling_on_sc=True),
    scratch_types=[
        pltpu.VMEM((B_PER_W,), jnp.int32),
        pltpu.VMEM((B_PER_W,), jnp.int32),
        pltpu.VMEM((B_PER_W,), jnp.int32),
        pltpu.VMEM((B_PER_W,), jnp.int32),
        pltpu.VMEM((2, CHUNK, 128), jnp.float32),
        pltpu.VMEM((2, CHUNK, 128), jnp.float32),
        pltpu.VMEM((FACTOR, LANES), jnp.float32),
        pltpu.VMEM((LANES,), jnp.float32),
        pltpu.VMEM((B_PER_W,), jnp.float32),
        pltpu.SemaphoreType.DMA((2,)),
    ],
)(_gmf_body)


def kernel(user, item, embed_user_GMF, embed_item_GMF, predict_W, predict_b):
    user_r = user.astype(jnp.int32).reshape(NW, B_PER_W)
    item_r = item.astype(jnp.int32).reshape(NW, B_PER_W)
    tab_u = embed_user_GMF.reshape(PACKED_ROWS, PACK * FACTOR)
    tab_i = embed_item_GMF.reshape(PACKED_ROWS, PACK * FACTOR)
    w_b = jnp.broadcast_to(predict_W.reshape(FACTOR, 1), (FACTOR, LANES))
    b_b = jnp.broadcast_to(predict_b.reshape(1), (LANES,))
    return _gmf(user_r, item_r, tab_u, tab_i, w_b, b_b)


# speedup vs baseline: 17.2990x; 3.0874x over previous
import functools

import jax
import jax.numpy as jnp
from jax import lax
from jax.experimental import pallas as pl
from jax.experimental.pallas import tpu as pltpu
from jax.experimental.pallas import tpu_sc as plsc

FACTOR = 32
BATCH = 16384
LANES = 16
BLOCK = 128

NC, NS = 2, 16
NW = NC * NS
B_PER_W = BATCH // NW
NGROUP = B_PER_W // LANES


def _gmf_body(user_hbm, item_hbm, tab_u, tab_i, w_hbm, b_hbm, out_hbm,
              idx_u, idx_i, slabs, vals_u, vals_i, w_v, b_v, out_v, sems):
    wid = lax.axis_index("s") * NC + lax.axis_index("c")
    base = wid * B_PER_W

    pltpu.sync_copy(user_hbm.at[wid], idx_u)
    pltpu.sync_copy(item_hbm.at[wid], idx_i)
    pltpu.sync_copy(w_hbm, w_v)
    pltpu.sync_copy(b_hbm, b_v)

    lidx = lax.iota(jnp.int32, LANES)
    bias = b_v[...]

    def gather_pass(tab, idx, vals):
        def fire(g):
            vec = idx[pl.ds(g * LANES, LANES)]
            off = lax.shift_right_logical(vec, 7) * BLOCK
            for k in range(LANES):
                pltpu.async_copy(
                    tab.at[:, pl.ds(pl.multiple_of(off[k], BLOCK), BLOCK)],
                    slabs.at[k], sems.at[k])

        def drain():
            for k in range(LANES):
                pltpu.make_async_copy(
                    tab.at[:, pl.ds(0, BLOCK)], slabs.at[k],
                    sems.at[k]).wait()

        def extract(g):
            vec = idx[pl.ds(g * LANES, LANES)]
            lane = vec & (BLOCK - 1)
            for k in range(LANES):
                e = g * LANES + k
                cl = jnp.full((LANES,), lane[k], jnp.int32)
                ks = jnp.full((LANES,), k, jnp.int32)
                ce = jnp.full((LANES,), e, jnp.int32)
                lo = plsc.load_gather(slabs, [ks, lidx, cl])
                hi = plsc.load_gather(slabs, [ks, lidx + LANES, cl])
                plsc.store_scatter(vals, [lidx, ce], lo)
                plsc.store_scatter(vals, [lidx + LANES, ce], hi)

        def group(g, carry):
            fire(g)
            drain()
            extract(g)
            return carry

        lax.fori_loop(0, NGROUP, group, 0)

    gather_pass(tab_u, idx_u, vals_u)
    gather_pass(tab_i, idx_i, vals_i)

    def compute(g, carry):
        db = pl.ds(g * LANES, LANES)
        acc = bias
        for f in range(FACTOR):
            acc = acc + vals_u[f, db] * vals_i[f, db] * w_v[f, :]
        out_v[db] = acc
        return carry

    lax.fori_loop(0, NGROUP, compute, 0)
    pltpu.sync_copy(out_v, out_hbm.at[pl.ds(base, B_PER_W)])


_gmf = functools.partial(
    pl.kernel,
    mesh=plsc.VectorSubcoreMesh(
        core_axis_name="c", subcore_axis_name="s",
        num_cores=NC, num_subcores=NS),
    out_type=jax.ShapeDtypeStruct((BATCH,), jnp.float32),
    compiler_params=pltpu.CompilerParams(
        needs_layout_passes=False, use_tc_tiling_on_sc=True,
        disable_bounds_checks=True),
    scratch_types=[
        pltpu.VMEM((B_PER_W,), jnp.int32),
        pltpu.VMEM((B_PER_W,), jnp.int32),
        pltpu.VMEM((LANES, FACTOR, BLOCK), jnp.float32),
        pltpu.VMEM((FACTOR, B_PER_W), jnp.float32),
        pltpu.VMEM((FACTOR, B_PER_W), jnp.float32),
        pltpu.VMEM((FACTOR, LANES), jnp.float32),
        pltpu.VMEM((LANES,), jnp.float32),
        pltpu.VMEM((B_PER_W,), jnp.float32),
        pltpu.SemaphoreType.DMA((LANES,)),
    ],
)(_gmf_body)


def kernel(user, item, embed_user_GMF, embed_item_GMF, predict_W, predict_b):
    user_r = user.astype(jnp.int32).reshape(NW, B_PER_W)
    item_r = item.astype(jnp.int32).reshape(NW, B_PER_W)
    w_b = jnp.broadcast_to(predict_W.reshape(FACTOR, 1), (FACTOR, LANES))
    b_b = jnp.broadcast_to(predict_b.reshape(1), (LANES,))
    return _gmf(user_r, item_r, embed_user_GMF.T, embed_item_GMF.T, w_b, b_b)


# speedup vs baseline: 20.6921x; 1.1961x over previous
import functools

import jax
import jax.numpy as jnp
from jax import lax
from jax.experimental import pallas as pl
from jax.experimental.pallas import tpu as pltpu
from jax.experimental.pallas import tpu_sc as plsc

FACTOR = 32
BATCH = 16384
LANES = 16
BLOCK = 128

NC, NS = 2, 16
NW = NC * NS
B_PER_W = BATCH // NW
NGROUP = B_PER_W // LANES


def _gmf_body(user_hbm, item_hbm, tab_u, tab_i, w_hbm, b_hbm, out_hbm,
              idx_u, idx_i, slabs, vals_u, vals_i, w_v, b_v, out_v, sems):
    wid = lax.axis_index("s") * NC + lax.axis_index("c")
    base = wid * B_PER_W

    pltpu.sync_copy(user_hbm.at[wid], idx_u)
    pltpu.sync_copy(item_hbm.at[wid], idx_i)
    pltpu.sync_copy(w_hbm, w_v)
    pltpu.sync_copy(b_hbm, b_v)

    lidx = lax.iota(jnp.int32, LANES)
    bias = b_v[...]

    HALF = LANES // 2

    def gather_pass(tab, idx, vals):
        def fire(g16, par):
            vec = idx[pl.ds(g16 * LANES, LANES)]
            off = lax.shift_right_logical(vec, 7) * BLOCK
            for k in range(HALF):
                kk = par * HALF + k
                pltpu.async_copy(
                    tab.at[:, pl.ds(pl.multiple_of(off[kk], BLOCK), BLOCK)],
                    slabs.at[kk], sems.at[kk])

        def drain(par):
            for k in range(HALF):
                s = par * HALF + k
                pltpu.make_async_copy(
                    tab.at[:, pl.ds(0, BLOCK)], slabs.at[s],
                    sems.at[s]).wait()

        def extract(g16, par):
            vec = idx[pl.ds(g16 * LANES, LANES)]
            lane = vec & (BLOCK - 1)
            for k in range(HALF):
                kk = par * HALF + k
                e = g16 * LANES + kk
                cl = jnp.full((LANES,), lane[kk], jnp.int32)
                ks = jnp.full((LANES,), kk, jnp.int32)
                ce = jnp.full((LANES,), e, jnp.int32)
                lo = plsc.load_gather(slabs, [ks, lidx, cl])
                hi = plsc.load_gather(slabs, [ks, lidx + LANES, cl])
                plsc.store_scatter(vals, [lidx, ce], lo)
                plsc.store_scatter(vals, [lidx + LANES, ce], hi)

        fire(0, 0)
        fire(0, 1)

        def pair(p, carry):
            drain(0)
            extract(p, 0)

            @pl.when(p + 1 < NGROUP)
            def _():
                fire(p + 1, 0)

            drain(1)
            extract(p, 1)

            @pl.when(p + 1 < NGROUP)
            def _():
                fire(p + 1, 1)

            return carry

        lax.fori_loop(0, NGROUP, pair, 0)

    gather_pass(tab_u, idx_u, vals_u)
    gather_pass(tab_i, idx_i, vals_i)

    def compute(g, carry):
        db = pl.ds(g * LANES, LANES)
        acc = bias
        for f in range(FACTOR):
            acc = acc + vals_u[f, db] * vals_i[f, db] * w_v[f, :]
        out_v[db] = acc
        return carry

    lax.fori_loop(0, NGROUP, compute, 0)
    pltpu.sync_copy(out_v, out_hbm.at[pl.ds(base, B_PER_W)])


_gmf = functools.partial(
    pl.kernel,
    mesh=plsc.VectorSubcoreMesh(
        core_axis_name="c", subcore_axis_name="s",
        num_cores=NC, num_subcores=NS),
    out_type=jax.ShapeDtypeStruct((BATCH,), jnp.float32),
    compiler_params=pltpu.CompilerParams(
        needs_layout_passes=False, use_tc_tiling_on_sc=True,
        disable_bounds_checks=True),
    scratch_types=[
        pltpu.VMEM((B_PER_W,), jnp.int32),
        pltpu.VMEM((B_PER_W,), jnp.int32),
        pltpu.VMEM((LANES, FACTOR, BLOCK), jnp.float32),
        pltpu.VMEM((FACTOR, B_PER_W), jnp.float32),
        pltpu.VMEM((FACTOR, B_PER_W), jnp.float32),
        pltpu.VMEM((FACTOR, LANES), jnp.float32),
        pltpu.VMEM((LANES,), jnp.float32),
        pltpu.VMEM((B_PER_W,), jnp.float32),
        pltpu.SemaphoreType.DMA((LANES,)),
    ],
)(_gmf_body)


def kernel(user, item, embed_user_GMF, embed_item_GMF, predict_W, predict_b):
    user_r = user.astype(jnp.int32).reshape(NW, B_PER_W)
    item_r = item.astype(jnp.int32).reshape(NW, B_PER_W)
    w_b = jnp.broadcast_to(predict_W.reshape(FACTOR, 1), (FACTOR, LANES))
    b_b = jnp.broadcast_to(predict_b.reshape(1), (LANES,))
    return _gmf(user_r, item_r, embed_user_GMF.T, embed_item_GMF.T, w_b, b_b)
